# baseline (speedup 1.0000x reference)
import functools

import jax
import jax.numpy as jnp
from jax.experimental import pallas as pl

_NN = 10
_SIGMA = 5.0
_BIG = 3.0e38
_PAD_COORD = 1.0e18
_LANES = 128
_NCHUNK = 3


def _nnencode_block(pts_ref, cc_ref, out_ref):
    pts = pts_ref[0]
    x = pts[:, 0:1]
    y = pts[:, 1:2]
    p2 = x * x + y * y

    d2s = []
    for j in range(_NCHUNK):
        cx = cc_ref[0:1, j * _LANES:(j + 1) * _LANES]
        cy = cc_ref[1:2, j * _LANES:(j + 1) * _LANES]
        c2 = cx * cx + cy * cy
        cross = x * cx + y * cy
        d2s.append(jnp.maximum(p2 + c2 - 2.0 * cross, 0.0))
    d0, d1, d2c = d2s

    lo = jnp.minimum(d0, d1)
    hi = jnp.maximum(d0, d1)
    a = jnp.minimum(lo, d2c)
    b = jnp.maximum(lo, jnp.minimum(hi, d2c))
    c = jnp.maximum(hi, d2c)

    thr = None
    for _ in range(_NN):
        thr = jnp.min(a, axis=1, keepdims=True)
        pop = a <= thr
        a = jnp.where(pop, b, a)
        b = jnp.where(pop, c, b)
        c = jnp.where(pop, _BIG, c)

    scale = -1.0 / (2.0 * _SIGMA ** 2)
    ws = []
    keeps = []
    s = None
    for j in range(_NCHUNK):
        keep = d2s[j] <= thr
        w = jnp.where(keep, jnp.exp(d2s[j] * scale), 0.0)
        keeps.append(keep)
        ws.append(w)
        sj = jnp.sum(w, axis=1, keepdims=True)
        s = sj if s is None else s + sj

    inv = 1.0 / s
    full = jnp.concatenate(
        [jnp.where(keeps[j], ws[j] * inv, 0.0) for j in range(_NCHUNK)], axis=1)
    out_ref[0] = full[:, :out_ref.shape[2]]


@functools.partial(jax.jit, static_argnames=("block_rows", "interpret"))
def _nnencode(pts_nd, cc, block_rows=64, interpret=False):
    B, S, D = pts_nd.shape
    K = cc.shape[0]
    kp = _NCHUNK * _LANES
    cc_pad = jnp.full((D, kp), _PAD_COORD, dtype=cc.dtype).at[:, :K].set(cc.T)
    grid = (B, S // block_rows)
    return pl.pallas_call(
        _nnencode_block,
        grid=grid,
        in_specs=[
            pl.BlockSpec((1, block_rows, D), lambda i, j: (i, j, 0)),
            pl.BlockSpec((D, kp), lambda i, j: (0, 0)),
        ],
        out_specs=pl.BlockSpec((1, block_rows, K), lambda i, j: (i, j, 0)),
        out_shape=jax.ShapeDtypeStruct((B, S, K), jnp.float32),
        interpret=interpret,
    )(pts_nd, cc_pad)


def kernel(pts_nd, cc):
    return _nnencode(pts_nd, cc)


# speedup vs baseline: 2.8653x; 2.8653x over previous
import functools

import jax
import jax.numpy as jnp
from jax.experimental import pallas as pl

_NN = 10
_SIGMA = 5.0
_BIG = 3.0e38
_PAD_COORD = 1.0e18
_LANES = 128
_NCHUNK = 3


def _nnencode_block(pts_ref, cc_ref, out_ref):
    pts = pts_ref[0]
    x = pts[:, 0:1]
    y = pts[:, 1:2]
    p2 = x * x + y * y

    d2s = []
    for j in range(_NCHUNK):
        cx = cc_ref[0:1, j * _LANES:(j + 1) * _LANES]
        cy = cc_ref[1:2, j * _LANES:(j + 1) * _LANES]
        c2 = cx * cx + cy * cy
        cross = x * cx + y * cy
        d2s.append(jnp.maximum(p2 + c2 - 2.0 * cross, 0.0))
    d0, d1, d2c = d2s

    lo = jnp.minimum(d0, d1)
    hi = jnp.maximum(d0, d1)
    a = jnp.minimum(lo, d2c)
    b = jnp.maximum(lo, jnp.minimum(hi, d2c))
    c = jnp.maximum(hi, d2c)

    thr = None
    for _ in range(_NN):
        thr = jnp.min(a, axis=1, keepdims=True)
        pop = a <= thr
        a = jnp.where(pop, b, a)
        b = jnp.where(pop, c, b)
        c = jnp.where(pop, _BIG, c)

    scale = -1.0 / (2.0 * _SIGMA ** 2)
    ws = []
    keeps = []
    s = None
    for j in range(_NCHUNK):
        keep = d2s[j] <= thr
        w = jnp.where(keep, jnp.exp(d2s[j] * scale), 0.0)
        keeps.append(keep)
        ws.append(w)
        sj = jnp.sum(w, axis=1, keepdims=True)
        s = sj if s is None else s + sj

    inv = 1.0 / s
    full = jnp.concatenate(
        [jnp.where(keeps[j], ws[j] * inv, 0.0) for j in range(_NCHUNK)], axis=1)
    out_ref[0] = full[:, :out_ref.shape[2]]


@functools.partial(jax.jit, static_argnames=("block_rows", "interpret"))
def _nnencode(pts_nd, cc, block_rows=256, interpret=False):
    B, S, D = pts_nd.shape
    K = cc.shape[0]
    kp = _NCHUNK * _LANES
    cc_pad = jnp.full((D, kp), _PAD_COORD, dtype=cc.dtype).at[:, :K].set(cc.T)
    grid = (B, S // block_rows)
    return pl.pallas_call(
        _nnencode_block,
        grid=grid,
        in_specs=[
            pl.BlockSpec((1, block_rows, D), lambda i, j: (i, j, 0)),
            pl.BlockSpec((D, kp), lambda i, j: (0, 0)),
        ],
        out_specs=pl.BlockSpec((1, block_rows, K), lambda i, j: (i, j, 0)),
        out_shape=jax.ShapeDtypeStruct((B, S, K), jnp.float32),
        interpret=interpret,
    )(pts_nd, cc_pad)


def kernel(pts_nd, cc):
    return _nnencode(pts_nd, cc)


# speedup vs baseline: 3.8867x; 1.3565x over previous
import functools

import jax
import jax.numpy as jnp
from jax.experimental import pallas as pl

_NN = 10
_SIGMA = 5.0
_BIG = 3.0e38


def _nnencode_block(pts_ref, cc_ref, out_ref):
    pts = pts_ref[0]
    x = pts[:, 0:1]
    y = pts[:, 1:2]
    cx = cc_ref[0:1, :]
    cy = cc_ref[1:2, :]
    p2 = x * x + y * y
    c2 = cx * cx + cy * cy
    cross = x * cx + y * cy
    d2 = jnp.maximum(p2 + c2 - 2.0 * cross, 0.0)

    cur = d2
    thr = None
    for _ in range(_NN):
        thr = jnp.min(cur, axis=1, keepdims=True)
        cur = jnp.where(cur <= thr, _BIG, cur)

    keep = d2 <= thr
    w = jnp.where(keep, jnp.exp(d2 * (-1.0 / (2.0 * _SIGMA ** 2))), 0.0)
    s = jnp.sum(w, axis=1, keepdims=True)
    out_ref[0] = jnp.where(keep, w / s, 0.0)


@functools.partial(jax.jit, static_argnames=("block_rows", "interpret"))
def _nnencode(pts_nd, cc, block_rows=2048, interpret=False):
    B, S, D = pts_nd.shape
    K = cc.shape[0]
    cc_t = cc.T
    grid = (B, S // block_rows)
    return pl.pallas_call(
        _nnencode_block,
        grid=grid,
        in_specs=[
            pl.BlockSpec((1, block_rows, D), lambda i, j: (i, j, 0)),
            pl.BlockSpec((D, K), lambda i, j: (0, 0)),
        ],
        out_specs=pl.BlockSpec((1, block_rows, K), lambda i, j: (i, j, 0)),
        out_shape=jax.ShapeDtypeStruct((B, S, K), jnp.float32),
        interpret=interpret,
    )(pts_nd, cc_t)


def kernel(pts_nd, cc):
    return _nnencode(pts_nd, cc)
